# Initial kernel scaffold; baseline (speedup 1.0000x reference)
import jax
import jax.numpy as jnp
from jax.experimental import pallas as pl


def kernel(x, mask, weight, bias):
    raise NotImplementedError("write your pallas kernel here")


# speedup vs baseline: 7.3277x; 7.3277x over previous
import jax
import jax.numpy as jnp
from jax.experimental import pallas as pl
from jax.experimental.pallas import tpu as pltpu

_B, _C, _H, _W = 4, 16, 32, 32
_K = 3
_NC = 32
_D = _C * _K * _K
_N = _B * _H * _W
_BN = 128
_ITERS = 26


def _unfold_rows(x):
    pb = (_K - 1) // 2
    pe = (_K - 1) - pb
    xp = jnp.pad(x, ((0, 0), (0, 0), (pb, pe), (pb, pe)))
    hout = xp.shape[2] - _K + 1
    wout = xp.shape[3] - _K + 1
    patches = jnp.stack([xp[:, :, i:i + hout, j:j + wout]
                         for i in range(_K) for j in range(_K)], axis=2)
    u = patches.reshape(x.shape[0], _D, hout * wout)
    return jnp.transpose(u, (0, 2, 1)).reshape(-1, _D)


def _wos_block(inp_ref, mask_ref, wp_ref, wn_ref, bias_ref, out_ref):
    inp = inp_ref[...]
    a = inp[:, None, :] + mask_ref[...][None, :, :]
    wp = wp_ref[...][None, :, :]
    wn = wn_ref[...][None, :, :]
    bias = bias_ref[...]
    m = jnp.max(jnp.abs(a), axis=2)
    delta = jnp.float32(1e-3)
    lo0 = -m - delta
    hi0 = m + delta

    def body(_, carry):
        lo, hi = carry
        mid = 0.5 * (lo + hi)
        midb = mid[:, :, None]
        f = jnp.sum(jnp.where(a >= midb, wp, 0.0)
                    + jnp.where(a <= -midb, wn, 0.0), axis=2)
        pred = f <= bias
        return jnp.where(pred, lo, mid), jnp.where(pred, mid, hi)

    lo, hi = jax.lax.fori_loop(0, _ITERS, body, (lo0, hi0))
    th = hi[:, :, None]
    big = jnp.float32(3.0e38)
    candp = jnp.where(a >= th, a, big)
    candn = jnp.where(a <= -th, -a, big)
    r = jnp.min(jnp.minimum(candp, candn), axis=2)
    r = jnp.where(r > jnp.float32(1e38), m, r)
    out_ref[...] = r


def kernel(x, mask, weight, bias):
    inp = _unfold_rows(x)
    wp = weight[:, :_D]
    wn = weight[:, _D:]
    bias_t = bias.reshape(1, _NC)
    out = pl.pallas_call(
        _wos_block,
        grid=(_N // _BN,),
        in_specs=[
            pl.BlockSpec((_BN, _D), lambda i: (i, 0)),
            pl.BlockSpec((_NC, _D), lambda i: (0, 0)),
            pl.BlockSpec((_NC, _D), lambda i: (0, 0)),
            pl.BlockSpec((_NC, _D), lambda i: (0, 0)),
            pl.BlockSpec((1, _NC), lambda i: (0, 0)),
        ],
        out_specs=pl.BlockSpec((_BN, _NC), lambda i: (i, 0)),
        out_shape=jax.ShapeDtypeStruct((_N, _NC), jnp.float32),
        compiler_params=pltpu.CompilerParams(
            dimension_semantics=("parallel",),
        ),
    )(inp, mask, wp, wn, bias_t)
    return out.reshape(-1, _NC, _H, _W)


# speedup vs baseline: 21.3880x; 2.9188x over previous
import jax
import jax.numpy as jnp
from jax.experimental import pallas as pl
from jax.experimental.pallas import tpu as pltpu

_B, _C, _H, _W = 4, 16, 32, 32
_K = 3
_NC = 32
_D = _C * _K * _K
_MD = 2 * _D
_N = _B * _H * _W
_BN = 512
_ITERS = 26


def _unfold_cols(x):
    pb = (_K - 1) // 2
    pe = (_K - 1) - pb
    xp = jnp.pad(x, ((0, 0), (0, 0), (pb, pe), (pb, pe)))
    hout = xp.shape[2] - _K + 1
    wout = xp.shape[3] - _K + 1
    patches = jnp.stack([xp[:, :, i:i + hout, j:j + wout]
                         for i in range(_K) for j in range(_K)], axis=2)
    u = patches.reshape(x.shape[0], _D, hout * wout)
    return jnp.transpose(u, (1, 0, 2)).reshape(_D, -1)


def _wos_block(inp_ref, mask_ref, w_ref, bias_ref, out_ref):
    a = inp_ref[...] + mask_ref[...].reshape(_D, 1)
    v = jnp.concatenate([a, -a], axis=0)
    w = w_ref[...].reshape(_MD, 1)
    bias = bias_ref[...].reshape(1, 1)
    m = jnp.max(v, axis=0, keepdims=True)
    delta = jnp.float32(1e-3)
    lo0 = -m - delta
    hi0 = m + delta

    def body(_, carry):
        lo, hi = carry
        mid = 0.5 * (lo + hi)
        f = jnp.sum(jnp.where(v >= mid, w, 0.0), axis=0, keepdims=True)
        pred = f <= bias
        return jnp.where(pred, lo, mid), jnp.where(pred, mid, hi)

    lo, hi = jax.lax.fori_loop(0, _ITERS, body, (lo0, hi0))
    big = jnp.float32(3.0e38)
    r = jnp.min(jnp.where(v >= hi, v, big), axis=0, keepdims=True)
    r = jnp.where(r > jnp.float32(1e38), m, r)
    out_ref[...] = r.reshape(1, 1, _BN)


def kernel(x, mask, weight, bias):
    inp_t = _unfold_cols(x)
    mask_r = mask.reshape(_NC, _D, 1)
    weight_r = weight.reshape(_NC, _MD, 1)
    bias_r = bias.reshape(_NC, 1, 1)
    out = pl.pallas_call(
        _wos_block,
        grid=(_N // _BN, _NC),
        in_specs=[
            pl.BlockSpec((_D, _BN), lambda i, nc: (0, i)),
            pl.BlockSpec((1, _D, 1), lambda i, nc: (nc, 0, 0)),
            pl.BlockSpec((1, _MD, 1), lambda i, nc: (nc, 0, 0)),
            pl.BlockSpec((1, 1, 1), lambda i, nc: (nc, 0, 0)),
        ],
        out_specs=pl.BlockSpec((1, 1, _BN), lambda i, nc: (nc, 0, i)),
        out_shape=jax.ShapeDtypeStruct((_NC, 1, _N), jnp.float32),
        compiler_params=pltpu.CompilerParams(
            dimension_semantics=("parallel", "arbitrary"),
        ),
    )(inp_t, mask_r, weight_r, bias_r)
    y = out.reshape(_NC, _N).T
    return y.reshape(-1, _NC, _H, _W)


# speedup vs baseline: 30.6149x; 1.4314x over previous
import jax
import jax.numpy as jnp
from jax.experimental import pallas as pl
from jax.experimental.pallas import tpu as pltpu

_B, _C, _H, _W = 4, 16, 32, 32
_K = 3
_NC = 32
_D = _C * _K * _K
_MD = 2 * _D
_N = _B * _H * _W
_BN = 4096
_ITERS = 26


def _unfold_cols(x):
    pb = (_K - 1) // 2
    pe = (_K - 1) - pb
    xp = jnp.pad(x, ((0, 0), (0, 0), (pb, pe), (pb, pe)))
    hout = xp.shape[2] - _K + 1
    wout = xp.shape[3] - _K + 1
    patches = jnp.stack([xp[:, :, i:i + hout, j:j + wout]
                         for i in range(_K) for j in range(_K)], axis=2)
    u = patches.reshape(x.shape[0], _D, hout * wout)
    return jnp.transpose(u, (1, 0, 2)).reshape(_D, -1)


def _wos_block(inp_ref, mask_ref, w_ref, bias_ref, out_ref):
    a = inp_ref[...] + mask_ref[...].reshape(_D, 1)
    v = jnp.concatenate([a, -a], axis=0)
    w = w_ref[...].reshape(_MD, 1)
    bias = bias_ref[...].reshape(1, 1)
    m = jnp.max(v, axis=0, keepdims=True)
    delta = jnp.float32(1e-3)
    lo0 = -m - delta
    hi0 = m + delta

    def body(_, carry):
        lo, hi = carry
        mid = 0.5 * (lo + hi)
        f = jnp.sum(jnp.where(v >= mid, w, 0.0), axis=0, keepdims=True)
        pred = f <= bias
        return jnp.where(pred, lo, mid), jnp.where(pred, mid, hi)

    lo, hi = jax.lax.fori_loop(0, _ITERS, body, (lo0, hi0))
    big = jnp.float32(3.0e38)
    r = jnp.min(jnp.where(v >= hi, v, big), axis=0, keepdims=True)
    r = jnp.where(r > jnp.float32(1e38), m, r)
    out_ref[...] = r.reshape(1, 1, _BN)


def kernel(x, mask, weight, bias):
    inp_t = _unfold_cols(x)
    mask_r = mask.reshape(_NC, _D, 1)
    weight_r = weight.reshape(_NC, _MD, 1)
    bias_r = bias.reshape(_NC, 1, 1)
    out = pl.pallas_call(
        _wos_block,
        grid=(_NC,),
        in_specs=[
            pl.BlockSpec((_D, _BN), lambda nc: (0, 0)),
            pl.BlockSpec((1, _D, 1), lambda nc: (nc, 0, 0)),
            pl.BlockSpec((1, _MD, 1), lambda nc: (nc, 0, 0)),
            pl.BlockSpec((1, 1, 1), lambda nc: (nc, 0, 0)),
        ],
        out_specs=pl.BlockSpec((1, 1, _BN), lambda nc: (nc, 0, 0)),
        out_shape=jax.ShapeDtypeStruct((_NC, 1, _N), jnp.float32),
        compiler_params=pltpu.CompilerParams(
            dimension_semantics=("parallel",),
        ),
    )(inp_t, mask_r, weight_r, bias_r)
    y = out.reshape(_NC, _N).T
    return y.reshape(-1, _NC, _H, _W)


# speedup vs baseline: 45.8866x; 1.4988x over previous
import jax
import jax.numpy as jnp
from jax.experimental import pallas as pl
from jax.experimental.pallas import tpu as pltpu

_B, _C, _H, _W = 4, 16, 32, 32
_K = 3
_NC = 32
_D = _C * _K * _K
_MD = 2 * _D
_N = _B * _H * _W
_BN = 4096
_ITERS = 26


def _unfold_cols(x):
    pb = (_K - 1) // 2
    pe = (_K - 1) - pb
    xp = jnp.pad(x, ((0, 0), (0, 0), (pb, pe), (pb, pe)))
    hout = xp.shape[2] - _K + 1
    wout = xp.shape[3] - _K + 1
    patches = jnp.stack([xp[:, :, i:i + hout, j:j + wout]
                         for i in range(_K) for j in range(_K)], axis=2)
    u = patches.reshape(x.shape[0], _D, hout * wout)
    return jnp.transpose(u, (1, 0, 2)).reshape(_D, -1)


def _wos_block(inp_ref, mask_ref, w_ref, bias_ref, out_ref):
    a = inp_ref[...] + mask_ref[...].reshape(_D, 1)
    v = jnp.concatenate([a, -a], axis=0)
    w = w_ref[...].reshape(1, _MD)
    bias = bias_ref[...].reshape(1, 1)
    m = jnp.max(v, axis=0, keepdims=True)
    delta = jnp.float32(1e-3)
    lo0 = -m - delta
    hi0 = m + delta

    def body(_, carry):
        lo, hi = carry
        mid = 0.5 * (lo + hi)
        ind = jnp.where(v >= mid, 1.0, 0.0)
        f = jnp.dot(w, ind, preferred_element_type=jnp.float32)
        pred = f <= bias
        return jnp.where(pred, lo, mid), jnp.where(pred, mid, hi)

    lo, hi = jax.lax.fori_loop(0, _ITERS, body, (lo0, hi0))
    big = jnp.float32(3.0e38)
    r = jnp.min(jnp.where(v >= hi, v, big), axis=0, keepdims=True)
    r = jnp.where(r > jnp.float32(1e38), m, r)
    out_ref[...] = r.reshape(1, 1, _BN)


def kernel(x, mask, weight, bias):
    inp_t = _unfold_cols(x)
    mask_r = mask.reshape(_NC, _D, 1)
    weight_r = weight.reshape(_NC, 1, _MD)
    bias_r = bias.reshape(_NC, 1, 1)
    out = pl.pallas_call(
        _wos_block,
        grid=(_NC,),
        in_specs=[
            pl.BlockSpec((_D, _BN), lambda nc: (0, 0)),
            pl.BlockSpec((1, _D, 1), lambda nc: (nc, 0, 0)),
            pl.BlockSpec((1, 1, _MD), lambda nc: (nc, 0, 0)),
            pl.BlockSpec((1, 1, 1), lambda nc: (nc, 0, 0)),
        ],
        out_specs=pl.BlockSpec((1, 1, _BN), lambda nc: (nc, 0, 0)),
        out_shape=jax.ShapeDtypeStruct((_NC, 1, _N), jnp.float32),
        compiler_params=pltpu.CompilerParams(
            dimension_semantics=("parallel",),
        ),
    )(inp_t, mask_r, weight_r, bias_r)
    y = out.reshape(_NC, _N).T
    return y.reshape(-1, _NC, _H, _W)
